# baseline (speedup 1.0000x reference)
import dataclasses
import functools

import jax
import jax.numpy as jnp
from jax import lax
from jax.experimental import pallas as pl
from jax.experimental.pallas import tpu as pltpu
from jax.experimental.pallas import tpu_sc as plsc

_SC_CORES = 2
_SC_SUBCORES = 16
_SC_WORKERS = _SC_CORES * _SC_SUBCORES


def _argmin_body(total_k, x_ref, k_ref, o_ref, xsq_ref, minv_ref, argv_ref):
    c = pl.program_id(0)
    kt = pl.program_id(1)
    num_kt = pl.num_programs(1)
    x = x_ref[0]
    ks = k_ref[0]
    tk = ks.shape[0]

    @pl.when(kt == 0)
    def _():
        xsq_ref[...] = jnp.sum(x * x, axis=1, keepdims=True)

    dots = lax.dot_general(
        x, ks, (((1,), (1,)), ((), ())),
        precision=lax.Precision.DEFAULT,
        preferred_element_type=jnp.float32)
    ones = jnp.ones((1, ks.shape[1]), jnp.float32)
    k_sq = lax.dot_general(
        ones, ks * ks, (((1,), (1,)), ((), ())),
        precision=lax.Precision.HIGHEST,
        preferred_element_type=jnp.float32)
    dist = (xsq_ref[...] + k_sq) - 2.0 * dots
    tmin = jnp.min(dist, axis=1, keepdims=True)
    kiota = lax.broadcasted_iota(jnp.int32, dist.shape, 1) + kt * tk
    masked = jnp.where(dist == tmin, kiota, total_k)
    targ = jnp.min(masked, axis=1, keepdims=True)

    @pl.when(kt == 0)
    def _():
        minv_ref[...] = tmin
        argv_ref[...] = targ

    @pl.when(kt != 0)
    def _():
        better = tmin < minv_ref[...]
        argv_ref[...] = jnp.where(better, targ, argv_ref[...])
        minv_ref[...] = jnp.minimum(tmin, minv_ref[...])

    @pl.when(kt == num_kt - 1)
    def _():
        o_ref[0] = argv_ref[...] + c * total_k


def _make_sc_gather_reduce(nb, nc, vdim):
    b_per_w = nb // _SC_WORKERS
    rows_per_w = b_per_w * nc
    nchunk = vdim // 16
    mesh = plsc.VectorSubcoreMesh(core_axis_name="c", subcore_axis_name="s")
    cp = pltpu.CompilerParams()
    if "needs_layout_passes" in pltpu.CompilerParams.__dataclass_fields__:
        cp = dataclasses.replace(cp, needs_layout_passes=False)

    @functools.partial(
        pl.kernel, mesh=mesh,
        compiler_params=cp,
        out_type=jax.ShapeDtypeStruct((nb, vdim), jnp.float32),
        scratch_types=[
            pltpu.VMEM((rows_per_w,), jnp.int32),
            pltpu.VMEM((rows_per_w, vdim), jnp.float32),
            pltpu.VMEM((b_per_w, vdim), jnp.float32),
            pltpu.SemaphoreType.DMA,
        ],
    )
    def gather_kernel(table_hbm, idx_hbm, out_hbm, idx_v, rows_v, out_v, sem):
        wid = lax.axis_index("s") * _SC_CORES + lax.axis_index("c")
        pltpu.sync_copy(idx_hbm.at[pl.ds(wid * rows_per_w, rows_per_w)], idx_v)
        pltpu.async_copy(table_hbm.at[idx_v], rows_v, sem).wait()
        inv = jnp.float32(1.0 / nc)
        for bb in range(b_per_w):
            vals = []
            for t in range(nchunk):
                a = rows_v[nc * bb, pl.ds(16 * t, 16)]
                for r in range(1, nc):
                    a = a + rows_v[nc * bb + r, pl.ds(16 * t, 16)]
                vals.append(a * inv)
            m = vals[0]
            for t in range(1, nchunk):
                m = jnp.maximum(m, vals[t])
            mm = jnp.max(m)
            es = [jnp.exp(v - mm) for v in vals]
            sv = es[0]
            for t in range(1, nchunk):
                sv = sv + es[t]
            ss = jnp.sum(sv)
            for t in range(nchunk):
                out_v[bb, pl.ds(16 * t, 16)] = es[t] / ss
        pltpu.sync_copy(out_v, out_hbm.at[pl.ds(wid * b_per_w, b_per_w)])

    return gather_kernel


def kernel(batch, keys, values):
    B, C, D = batch.shape
    K = keys.shape[1]
    V = values.shape[-1]

    TK = 2048
    bt = jnp.transpose(batch, (1, 0, 2))
    idx = pl.pallas_call(
        functools.partial(_argmin_body, K),
        grid=(C, K // TK),
        in_specs=[
            pl.BlockSpec((1, B, D), lambda c, kt: (c, 0, 0)),
            pl.BlockSpec((1, TK, D), lambda c, kt: (c, kt, 0)),
        ],
        out_specs=pl.BlockSpec((1, B, 1), lambda c, kt: (c, 0, 0)),
        out_shape=jax.ShapeDtypeStruct((C, B, 1), jnp.int32),
        scratch_shapes=[
            pltpu.VMEM((B, 1), jnp.float32),
            pltpu.VMEM((B, 1), jnp.float32),
            pltpu.VMEM((B, 1), jnp.int32),
        ],
    )(bt, keys)

    idx_bmajor = idx.reshape(C, B).T.reshape(C * B)
    table = values.reshape(C * K, V)
    return _make_sc_gather_reduce(B, C, V)(table, idx_bmajor)


# speedup vs baseline: 1.1141x; 1.1141x over previous
import dataclasses
import functools

import jax
import jax.numpy as jnp
from jax import lax
from jax.experimental import pallas as pl
from jax.experimental.pallas import tpu as pltpu
from jax.experimental.pallas import tpu_sc as plsc

_SC_CORES = 2
_SC_SUBCORES = 16
_SC_WORKERS = _SC_CORES * _SC_SUBCORES


def _argmin_body(total_k, x_ref, k_ref, ksq_ref, o_ref, xsq_ref, minv_ref,
                 argv_ref):
    c = pl.program_id(0)
    kt = pl.program_id(1)
    num_kt = pl.num_programs(1)
    x = x_ref[0]
    ks = k_ref[0]
    tk = ks.shape[0]

    @pl.when(kt == 0)
    def _():
        xsq_ref[...] = jnp.sum(x * x, axis=1, keepdims=True)

    dots = lax.dot_general(
        x.astype(jnp.bfloat16), ks.astype(jnp.bfloat16),
        (((1,), (1,)), ((), ())),
        preferred_element_type=jnp.float32)
    k_sq = ksq_ref[0]
    dist = (xsq_ref[...] + k_sq) - 2.0 * dots
    tmin = jnp.min(dist, axis=1, keepdims=True)
    kiota = lax.broadcasted_iota(jnp.int32, dist.shape, 1) + kt * tk
    masked = jnp.where(dist == tmin, kiota, total_k)
    targ = jnp.min(masked, axis=1, keepdims=True)

    @pl.when(kt == 0)
    def _():
        minv_ref[...] = tmin
        argv_ref[...] = targ

    @pl.when(kt != 0)
    def _():
        better = tmin < minv_ref[...]
        argv_ref[...] = jnp.where(better, targ, argv_ref[...])
        minv_ref[...] = jnp.minimum(tmin, minv_ref[...])

    @pl.when(kt == num_kt - 1)
    def _():
        o_ref[0] = argv_ref[...] + c * total_k


def _make_sc_gather_reduce(nb, nc, vdim):
    b_per_w = nb // _SC_WORKERS
    rows_per_w = b_per_w * nc
    nchunk = vdim // 16
    mesh = plsc.VectorSubcoreMesh(core_axis_name="c", subcore_axis_name="s")
    cp = pltpu.CompilerParams()
    if "needs_layout_passes" in pltpu.CompilerParams.__dataclass_fields__:
        cp = dataclasses.replace(cp, needs_layout_passes=False)

    @functools.partial(
        pl.kernel, mesh=mesh,
        compiler_params=cp,
        out_type=jax.ShapeDtypeStruct((nb, vdim), jnp.float32),
        scratch_types=[
            pltpu.VMEM((rows_per_w,), jnp.int32),
            pltpu.VMEM((rows_per_w, vdim), jnp.float32),
            pltpu.VMEM((b_per_w, vdim), jnp.float32),
            pltpu.SemaphoreType.DMA,
        ],
    )
    def gather_kernel(table_hbm, idx_hbm, out_hbm, idx_v, rows_v, out_v, sem):
        wid = lax.axis_index("s") * _SC_CORES + lax.axis_index("c")
        pltpu.sync_copy(idx_hbm.at[pl.ds(wid * rows_per_w, rows_per_w)], idx_v)
        pltpu.async_copy(table_hbm.at[idx_v], rows_v, sem).wait()
        inv = jnp.float32(1.0 / nc)
        for bb in range(b_per_w):
            vals = []
            for t in range(nchunk):
                a = rows_v[nc * bb, pl.ds(16 * t, 16)]
                for r in range(1, nc):
                    a = a + rows_v[nc * bb + r, pl.ds(16 * t, 16)]
                vals.append(a * inv)
            m = vals[0]
            for t in range(1, nchunk):
                m = jnp.maximum(m, vals[t])
            mm = jnp.max(m)
            es = [jnp.exp(v - mm) for v in vals]
            sv = es[0]
            for t in range(1, nchunk):
                sv = sv + es[t]
            ss = jnp.sum(sv)
            for t in range(nchunk):
                out_v[bb, pl.ds(16 * t, 16)] = es[t] / ss
        pltpu.sync_copy(out_v, out_hbm.at[pl.ds(wid * b_per_w, b_per_w)])

    return gather_kernel


def kernel(batch, keys, values):
    B, C, D = batch.shape
    K = keys.shape[1]
    V = values.shape[-1]

    TK = 2048
    bt = jnp.transpose(batch, (1, 0, 2))
    ksq = jnp.sum(keys * keys, axis=-1).reshape(C, 1, K)
    idx = pl.pallas_call(
        functools.partial(_argmin_body, K),
        grid=(C, K // TK),
        in_specs=[
            pl.BlockSpec((1, B, D), lambda c, kt: (c, 0, 0)),
            pl.BlockSpec((1, TK, D), lambda c, kt: (c, kt, 0)),
            pl.BlockSpec((1, 1, TK), lambda c, kt: (c, 0, kt)),
        ],
        out_specs=pl.BlockSpec((1, B, 1), lambda c, kt: (c, 0, 0)),
        out_shape=jax.ShapeDtypeStruct((C, B, 1), jnp.int32),
        scratch_shapes=[
            pltpu.VMEM((B, 1), jnp.float32),
            pltpu.VMEM((B, 1), jnp.float32),
            pltpu.VMEM((B, 1), jnp.int32),
        ],
    )(bt, keys, ksq)

    idx_bmajor = idx.reshape(C, B).T.reshape(C * B)
    table = values.reshape(C * K, V)
    return _make_sc_gather_reduce(B, C, V)(table, idx_bmajor)


# speedup vs baseline: 1.6276x; 1.4609x over previous
import dataclasses
import functools

import jax
import jax.numpy as jnp
from jax import lax
from jax.experimental import pallas as pl
from jax.experimental.pallas import tpu as pltpu
from jax.experimental.pallas import tpu_sc as plsc

_SC_CORES = 2
_SC_SUBCORES = 16
_SC_WORKERS = _SC_CORES * _SC_SUBCORES


def _argmin_body(total_k, x_ref, k_ref, ksq_ref, o_ref, xsq_ref, minv_ref,
                 argv_ref):
    c = pl.program_id(0)
    kt = pl.program_id(1)
    num_kt = pl.num_programs(1)
    x = x_ref[0]
    kb = k_ref[0]
    tk = kb.shape[1]

    @pl.when(kt == 0)
    def _():
        xsq_ref[...] = jnp.sum(x * x, axis=1, keepdims=True)

    dots = lax.dot_general(
        x.astype(jnp.bfloat16), kb,
        (((1,), (0,)), ((), ())),
        preferred_element_type=jnp.float32)
    dist = (xsq_ref[...] + ksq_ref[0]) - 2.0 * dots
    tmin = jnp.min(dist, axis=1, keepdims=True)
    kiota = lax.broadcasted_iota(jnp.int32, dist.shape, 1) + kt * tk
    masked = jnp.where(dist == tmin, kiota, total_k)
    targ = jnp.min(masked, axis=1, keepdims=True)

    @pl.when(kt == 0)
    def _():
        minv_ref[...] = tmin
        argv_ref[...] = targ

    @pl.when(kt != 0)
    def _():
        better = tmin < minv_ref[...]
        argv_ref[...] = jnp.where(better, targ, argv_ref[...])
        minv_ref[...] = jnp.minimum(tmin, minv_ref[...])

    @pl.when(kt == num_kt - 1)
    def _():
        o_ref[0] = argv_ref[...] + c * total_k


def _make_sc_gather_reduce(nb, nc, vdim):
    b_per_w = nb // _SC_WORKERS
    rows_per_w = b_per_w * nc
    nchunk = vdim // 16
    mesh = plsc.VectorSubcoreMesh(core_axis_name="c", subcore_axis_name="s")
    cp = pltpu.CompilerParams()
    if "needs_layout_passes" in pltpu.CompilerParams.__dataclass_fields__:
        cp = dataclasses.replace(cp, needs_layout_passes=False)

    @functools.partial(
        pl.kernel, mesh=mesh,
        compiler_params=cp,
        out_type=jax.ShapeDtypeStruct((nb, vdim), jnp.float32),
        scratch_types=[
            pltpu.VMEM((rows_per_w,), jnp.int32),
            pltpu.VMEM((rows_per_w, vdim), jnp.float32),
            pltpu.VMEM((b_per_w, vdim), jnp.float32),
            pltpu.SemaphoreType.DMA,
        ],
    )
    def gather_kernel(table_hbm, idx_hbm, out_hbm, idx_v, rows_v, out_v, sem):
        wid = lax.axis_index("s") * _SC_CORES + lax.axis_index("c")
        pltpu.sync_copy(idx_hbm.at[pl.ds(wid * rows_per_w, rows_per_w)], idx_v)
        pltpu.async_copy(table_hbm.at[idx_v], rows_v, sem).wait()
        inv = jnp.float32(1.0 / nc)
        for bb in range(b_per_w):
            vals = []
            for t in range(nchunk):
                a = rows_v[nc * bb, pl.ds(16 * t, 16)]
                for r in range(1, nc):
                    a = a + rows_v[nc * bb + r, pl.ds(16 * t, 16)]
                vals.append(a * inv)
            m = vals[0]
            for t in range(1, nchunk):
                m = jnp.maximum(m, vals[t])
            mm = jnp.max(m)
            es = [jnp.exp(v - mm) for v in vals]
            sv = es[0]
            for t in range(1, nchunk):
                sv = sv + es[t]
            ss = jnp.sum(sv)
            for t in range(nchunk):
                out_v[bb, pl.ds(16 * t, 16)] = es[t] / ss
        pltpu.sync_copy(out_v, out_hbm.at[pl.ds(wid * b_per_w, b_per_w)])

    return gather_kernel


def kernel(batch, keys, values):
    B, C, D = batch.shape
    K = keys.shape[1]
    V = values.shape[-1]
    TK = 2048

    bt = jnp.transpose(batch, (1, 0, 2))
    ksq = jnp.sum(keys * keys, axis=-1).reshape(C, 1, K)
    kbt = jnp.transpose(keys, (0, 2, 1)).astype(jnp.bfloat16)

    idx = pl.pallas_call(
        functools.partial(_argmin_body, K),
        grid=(C, K // TK),
        in_specs=[
            pl.BlockSpec((1, B, D), lambda c, kt: (c, 0, 0)),
            pl.BlockSpec((1, D, TK), lambda c, kt: (c, 0, kt)),
            pl.BlockSpec((1, 1, TK), lambda c, kt: (c, 0, kt)),
        ],
        out_specs=pl.BlockSpec((1, B, 1), lambda c, kt: (c, 0, 0)),
        out_shape=jax.ShapeDtypeStruct((C, B, 1), jnp.int32),
        scratch_shapes=[
            pltpu.VMEM((B, 1), jnp.float32),
            pltpu.VMEM((B, 1), jnp.float32),
            pltpu.VMEM((B, 1), jnp.int32),
        ],
    )(bt, kbt, ksq)

    idx_bmajor = idx.reshape(C, B).T.reshape(C * B)
    table = values.reshape(C * K, V)
    return _make_sc_gather_reduce(B, C, V)(table, idx_bmajor)


# speedup vs baseline: 2.2126x; 1.3594x over previous
import dataclasses
import functools

import jax
import jax.numpy as jnp
from jax import lax
from jax.experimental import pallas as pl
from jax.experimental.pallas import tpu as pltpu
from jax.experimental.pallas import tpu_sc as plsc

_SC_CORES = 2
_SC_SUBCORES = 16
_SC_WORKERS = _SC_CORES * _SC_SUBCORES


def _argmin_body(x_ref, k_ref, ksq_ref, o_ref):
    c = pl.program_id(0)
    x = x_ref[0]
    kb = k_ref[0]
    kdim = kb.shape[1]
    xsq = jnp.sum(x * x, axis=1, keepdims=True)
    dots = lax.dot_general(
        x.astype(jnp.bfloat16), kb,
        (((1,), (0,)), ((), ())),
        preferred_element_type=jnp.float32)
    dist = (xsq + ksq_ref[0]) - 2.0 * dots
    tmin = jnp.min(dist, axis=1, keepdims=True)
    kiota = lax.broadcasted_iota(jnp.int32, dist.shape, 1)
    masked = jnp.where(dist == tmin, kiota, kdim)
    first = jnp.min(masked, axis=1, keepdims=True)
    o_ref[0] = first + c * kdim


def _make_sc_gather_reduce(nb, nc, vdim):
    b_per_w = nb // _SC_WORKERS
    rows_per_w = b_per_w * nc
    nchunk = vdim // 16
    mesh = plsc.VectorSubcoreMesh(core_axis_name="c", subcore_axis_name="s")
    cp = pltpu.CompilerParams()
    if "needs_layout_passes" in pltpu.CompilerParams.__dataclass_fields__:
        cp = dataclasses.replace(cp, needs_layout_passes=False)

    @functools.partial(
        pl.kernel, mesh=mesh,
        compiler_params=cp,
        out_type=jax.ShapeDtypeStruct((nb, vdim), jnp.float32),
        scratch_types=[
            pltpu.VMEM((rows_per_w,), jnp.int32),
            pltpu.VMEM((rows_per_w, vdim), jnp.float32),
            pltpu.VMEM((b_per_w, vdim), jnp.float32),
            pltpu.SemaphoreType.DMA,
        ],
    )
    def gather_kernel(table_hbm, idx_hbm, out_hbm, idx_v, rows_v, out_v, sem):
        wid = lax.axis_index("s") * _SC_CORES + lax.axis_index("c")
        pltpu.sync_copy(idx_hbm.at[pl.ds(wid * rows_per_w, rows_per_w)], idx_v)
        pltpu.async_copy(table_hbm.at[idx_v], rows_v, sem).wait()
        inv = jnp.float32(1.0 / nc)
        for bb in range(b_per_w):
            vals = []
            for t in range(nchunk):
                a = rows_v[nc * bb, pl.ds(16 * t, 16)]
                for r in range(1, nc):
                    a = a + rows_v[nc * bb + r, pl.ds(16 * t, 16)]
                vals.append(a * inv)
            m = vals[0]
            for t in range(1, nchunk):
                m = jnp.maximum(m, vals[t])
            mm = jnp.max(m)
            es = [jnp.exp(v - mm) for v in vals]
            sv = es[0]
            for t in range(1, nchunk):
                sv = sv + es[t]
            ss = jnp.sum(sv)
            for t in range(nchunk):
                out_v[bb, pl.ds(16 * t, 16)] = es[t] / ss
        pltpu.sync_copy(out_v, out_hbm.at[pl.ds(wid * b_per_w, b_per_w)])

    return gather_kernel


def kernel(batch, keys, values):
    B, C, D = batch.shape
    K = keys.shape[1]
    V = values.shape[-1]
    TK = 2048

    bt = jnp.transpose(batch, (1, 0, 2))
    ksq = jnp.sum(keys * keys, axis=-1).reshape(C, 1, K)
    kbt = jnp.transpose(keys, (0, 2, 1)).astype(jnp.bfloat16)

    idx = pl.pallas_call(
        _argmin_body,
        grid=(C,),
        in_specs=[
            pl.BlockSpec((1, B, D), lambda c: (c, 0, 0)),
            pl.BlockSpec((1, D, K), lambda c: (c, 0, 0)),
            pl.BlockSpec((1, 1, K), lambda c: (c, 0, 0)),
        ],
        out_specs=pl.BlockSpec((1, B, 1), lambda c: (c, 0, 0)),
        out_shape=jax.ShapeDtypeStruct((C, B, 1), jnp.int32),
    )(bt, kbt, ksq)

    idx_bmajor = idx.reshape(C, B).T.reshape(C * B)
    table = values.reshape(C * K, V)
    return _make_sc_gather_reduce(B, C, V)(table, idx_bmajor)


# speedup vs baseline: 2.2143x; 1.0008x over previous
import dataclasses
import functools

import jax
import jax.numpy as jnp
from jax import lax
from jax.experimental import pallas as pl
from jax.experimental.pallas import tpu as pltpu
from jax.experimental.pallas import tpu_sc as plsc

_SC_CORES = 2
_SC_SUBCORES = 16
_SC_WORKERS = _SC_CORES * _SC_SUBCORES


def _argmin_body(x_ref, k_ref, ksq_ref, o_ref):
    c = pl.program_id(0)
    x = x_ref[0]
    kb = k_ref[0]
    kdim = kb.shape[1]
    xsq = jnp.sum(x * x, axis=1, keepdims=True)
    dots = lax.dot_general(
        x.astype(jnp.bfloat16), kb,
        (((1,), (0,)), ((), ())),
        preferred_element_type=jnp.float32)
    dist = (xsq + ksq_ref[0]) - 2.0 * dots
    tmin = jnp.min(dist, axis=1, keepdims=True)
    kiota = lax.broadcasted_iota(jnp.int32, dist.shape, 1)
    masked = jnp.where(dist == tmin, kiota, kdim)
    first = jnp.min(masked, axis=1, keepdims=True)
    o_ref[0] = first + c * kdim


def _make_sc_gather_reduce(nb, nc, vdim):
    b_per_w = nb // _SC_WORKERS
    rows_per_w = b_per_w * nc
    nchunk = vdim // 16
    mesh = plsc.VectorSubcoreMesh(core_axis_name="c", subcore_axis_name="s")
    cp = pltpu.CompilerParams()
    if "needs_layout_passes" in pltpu.CompilerParams.__dataclass_fields__:
        cp = dataclasses.replace(cp, needs_layout_passes=False)

    @functools.partial(
        pl.kernel, mesh=mesh,
        compiler_params=cp,
        out_type=jax.ShapeDtypeStruct((nb, vdim), jnp.float32),
        scratch_types=[
            pltpu.VMEM((rows_per_w,), jnp.int32),
            pltpu.VMEM((rows_per_w, vdim), jnp.float32),
            pltpu.VMEM((b_per_w, vdim), jnp.float32),
            pltpu.SemaphoreType.DMA,
        ],
    )
    def gather_kernel(table_hbm, idx_hbm, out_hbm, idx_v, rows_v, out_v, sem):
        wid = lax.axis_index("s") * _SC_CORES + lax.axis_index("c")
        pltpu.sync_copy(idx_hbm.at[pl.ds(wid * rows_per_w, rows_per_w)], idx_v)
        pltpu.async_copy(table_hbm.at[idx_v], rows_v, sem).wait()
        inv = jnp.float32(1.0 / nc)
        for bb in range(b_per_w):
            vals = []
            for t in range(nchunk):
                a = rows_v[nc * bb, pl.ds(16 * t, 16)]
                for r in range(1, nc):
                    a = a + rows_v[nc * bb + r, pl.ds(16 * t, 16)]
                vals.append(a * inv)
            m = vals[0]
            for t in range(1, nchunk):
                m = jnp.maximum(m, vals[t])
            mm = jnp.max(m)
            es = [jnp.exp(v - mm) for v in vals]
            sv = es[0]
            for t in range(1, nchunk):
                sv = sv + es[t]
            ss = jnp.sum(sv)
            for t in range(nchunk):
                out_v[bb, pl.ds(16 * t, 16)] = es[t] / ss
        pltpu.sync_copy(out_v, out_hbm.at[pl.ds(wid * b_per_w, b_per_w)])

    return gather_kernel


def kernel(batch, keys, values):
    B, C, D = batch.shape
    K = keys.shape[1]
    V = values.shape[-1]
    bt = jnp.transpose(batch, (1, 0, 2))
    ksq = jnp.sum(keys * keys, axis=-1).reshape(C, 1, K)
    kbt = jnp.transpose(keys, (0, 2, 1)).astype(jnp.bfloat16)

    idx = pl.pallas_call(
        _argmin_body,
        grid=(C,),
        in_specs=[
            pl.BlockSpec((1, B, D), lambda c: (c, 0, 0)),
            pl.BlockSpec((1, D, K), lambda c: (c, 0, 0)),
            pl.BlockSpec((1, 1, K), lambda c: (c, 0, 0)),
        ],
        out_specs=pl.BlockSpec((1, B, 1), lambda c: (c, 0, 0)),
        out_shape=jax.ShapeDtypeStruct((C, B, 1), jnp.int32),
    )(bt, kbt, ksq)

    idx_bmajor = idx.reshape(C, B).T.reshape(C * B)
    table = values.reshape(C * K, V)
    return _make_sc_gather_reduce(B, C, V)(table, idx_bmajor)
